# baseline (speedup 1.0000x reference)
import functools

import jax
import jax.numpy as jnp
from jax import lax
from jax.experimental import pallas as pl
from jax.experimental.pallas import tpu as pltpu
from jax.experimental.pallas import tpu_sc as plsc

H = 512
W = 512
C = 96
N_BOXES = 5000
GRID_PTS = 49
N_PAD = 5120
BN = 256
P_PAD = N_PAD * GRID_PTS
NC = 2
NS = 16
NW = NC * NS
CPT = P_PAD // NW
PB = 160
NBLK = CPT // PB
IB = PB * 4
IDMA = 128
NIDMA = IB // IDMA


def _idxw_body(boxes_ref, hw_ref, idx_ref, w_ref):
    b = boxes_ref[...]
    x1 = b[:, 0:1]
    y1 = b[:, 1:2]
    x2 = b[:, 2:3]
    y2 = b[:, 3:4]
    wm1 = hw_ref[0, 1]
    hm1 = hw_ref[0, 0]
    x1n = x1 / wm1
    y1n = y1 / hm1
    x2n = x2 / wm1
    y2n = y2 / hm1

    col = lax.broadcasted_iota(jnp.int32, (BN, 4 * GRID_PTS), 1)
    gy = col // 28
    gx = (col // 4) % 7
    jm = col % 4
    j0 = jm // 2
    j1 = jm % 2
    gyf = gy.astype(jnp.float32) / 6.0
    gxf = gx.astype(jnp.float32) / 6.0

    in_y = y1n * float(H - 1) + gyf * ((y2n - y1n) * float(H - 1))
    in_x = x1n * float(W - 1) + gxf * ((x2n - x1n) * float(W - 1))
    y0f = jnp.floor(in_y)
    x0f = jnp.floor(in_x)
    wy = in_y - y0f
    wx = in_x - x0f
    y0 = jnp.clip(y0f, 0.0, float(H - 1)).astype(jnp.int32)
    y1i = jnp.clip(y0f + 1.0, 0.0, float(H - 1)).astype(jnp.int32)
    x0 = jnp.clip(x0f, 0.0, float(W - 1)).astype(jnp.int32)
    x1i = jnp.clip(x0f + 1.0, 0.0, float(W - 1)).astype(jnp.int32)
    valid = (
        (in_y >= 0.0) & (in_y <= float(H - 1))
        & (in_x >= 0.0) & (in_x <= float(W - 1))
    )

    ysel = jnp.where(j0 == 0, y0, y1i)
    wys = jnp.where(j0 == 0, 1.0 - wy, wy)
    xsel = jnp.where(j1 == 0, x0, x1i)
    wxs = jnp.where(j1 == 0, 1.0 - wx, wx)

    idx_ref[...] = ysel * W + xsel
    w_ref[...] = jnp.where(valid, wys * wxs, 0.0)


def _compute_idx_w(boxes_pad, hw):
    return pl.pallas_call(
        _idxw_body,
        grid=(N_PAD // BN,),
        in_specs=[
            pl.BlockSpec((BN, 4), lambda i: (i, 0)),
            pl.BlockSpec(memory_space=pltpu.SMEM),
        ],
        out_specs=[
            pl.BlockSpec((BN, 4 * GRID_PTS), lambda i: (i, 0)),
            pl.BlockSpec((BN, 4 * GRID_PTS), lambda i: (i, 0)),
        ],
        out_shape=[
            jax.ShapeDtypeStruct((N_PAD, 4 * GRID_PTS), jnp.int32),
            jax.ShapeDtypeStruct((N_PAD, 4 * GRID_PTS), jnp.float32),
        ],
    )(boxes_pad, hw)


def _roi_body(img_ref, idx_ref, w_ref, out_ref, idx_v, w_v, rows_v, out_v, sem):
    wid = lax.axis_index("s") * NC + lax.axis_index("c")
    lane = lax.iota(jnp.int32, 16)

    def block(blk, carry):
        pt0 = wid * CPT + blk * PB
        pltpu.sync_copy(idx_ref.at[pl.ds(pt0 * 4, IB)], idx_v)
        pltpu.sync_copy(w_ref.at[pl.ds(pt0 * 4, IB)], w_v)
        descs = [
            pltpu.async_copy(
                img_ref.at[idx_v.at[pl.ds(j * IDMA, IDMA)]],
                rows_v.at[pl.ds(j * IDMA, IDMA)],
                sem,
            )
            for j in range(NIDMA)
        ]
        for d in descs:
            d.wait()

        def point(i, c):
            r = i * 4
            rsel = [jnp.full((16,), r + j, jnp.int32) for j in range(4)]
            ws = [plsc.load_gather(w_v, [rsel[j]]) for j in range(4)]
            orow = jnp.full((16,), i, jnp.int32)
            for k in range(C // 16):
                colk = lane + (16 * k)
                acc = ws[0] * plsc.load_gather(rows_v, [rsel[0], colk])
                for j in range(1, 4):
                    acc = acc + ws[j] * plsc.load_gather(rows_v, [rsel[j], colk])
                plsc.store_scatter(out_v, [orow, colk], acc)
            return c

        lax.fori_loop(0, PB, point, 0)
        pltpu.sync_copy(out_v, out_ref.at[pl.ds(pt0, PB)])
        return carry

    lax.fori_loop(0, NBLK, block, 0)


def _roi_gather(img2d, idx2d, w_flat):
    mesh = plsc.VectorSubcoreMesh(core_axis_name="c", subcore_axis_name="s")
    run = pl.kernel(
        _roi_body,
        out_type=jax.ShapeDtypeStruct((P_PAD, C), jnp.float32),
        scratch_types=[
            pltpu.VMEM((IB,), jnp.int32),
            pltpu.VMEM((IB,), jnp.float32),
            pltpu.VMEM((IB, C), jnp.float32),
            pltpu.VMEM((PB, C), jnp.float32),
            pltpu.SemaphoreType.DMA,
        ],
        mesh=mesh,
        compiler_params=pltpu.CompilerParams(
            needs_layout_passes=False, use_tc_tiling_on_sc=False
        ),
    )
    return run(img2d, idx2d, w_flat)


def kernel(metadata, image, boxes):
    boxes2d = boxes[0].astype(jnp.float32)
    boxes_pad = jnp.pad(boxes2d, ((0, N_PAD - N_BOXES), (0, 0)))
    hw = (metadata[0:1, 0:2] - 1.0).astype(jnp.float32)

    idx2d, w2d = _compute_idx_w(boxes_pad, hw)
    idx_flat = idx2d.reshape(P_PAD * 4)
    w_flat = w2d.reshape(P_PAD * 4)
    img2d = image[0].reshape(H * W, C)

    out = _roi_gather(img2d, idx_flat, w_flat)
    out = out[: N_BOXES * GRID_PTS]
    return out.reshape(1, N_BOXES, 7, 7, C)
